# baseline (speedup 1.0000x reference)
import functools

import jax
import jax.numpy as jnp
from jax import lax
from jax.experimental import pallas as pl
from jax.experimental.pallas import tpu as pltpu

_NCLASS = 64
_LONG = (2, 4)
_GPB = 2
_NH = 2


def _ta(lhs, rhs):
    return lax.dot_general(lhs, rhs, (((0,), (0,)), ((), ())),
                           preferred_element_type=jnp.float32)


def _tab(lhs, rhs):
    return lax.dot_general(lhs, rhs, (((0,), (1,)), ((), ())),
                           preferred_element_type=jnp.float32)


def _nn(lhs, rhs):
    return jnp.dot(lhs, rhs, preferred_element_type=jnp.float32)


def _ln_kernel(x_ref, s_ref, v_ref, vt_ref, r_ref,
               w1id_ref, w1sc_ref, w1l_ref, b1_ref,
               w2all_ref, w2l_ref, b2_ref,
               wm_ref, bm_ref, f1w_ref, f1b_ref, f2w_ref, f2b_ref,
               o_ref, zab_ref, *, n_long, nclass, gpb):
    bf = jnp.bfloat16
    F = b1_ref.shape[1]
    N = s_ref.shape[-1]
    Hn = N // _NH

    b1c = jnp.transpose(b1_ref[...], (1, 0))
    b2c = jnp.transpose(b2_ref[...], (1, 0))
    bm_row = bm_ref[...]
    f1w, f1b = f1w_ref[...], f1b_ref[...]
    f2w, f2b = f2w_ref[...], f2b_ref[...]

    for g in range(gpb):
        Xg = x_ref[g]
        Sg = s_ref[g]
        Vg = v_ref[g]
        Vtg = vt_ref[g]
        Rv = r_ref[g]
        zab = zab_ref.at[g]

        Rp = {1: Rv}
        cur = Rv
        for t in range(2, max(_LONG) + 1):
            cur = cur * Rv
            if t in _LONG:
                Rp[t] = cur

        def dvec(fw, fb, s):
            d = fb[0:1, s:s + 1]
            for ti, t in enumerate(_LONG):
                d = d + fw[ti:ti + 1, s:s + 1] * Rp[t]
            return d

        def spectral_pt(U, fw, fb, wl_ref):
            Gt = jnp.concatenate(
                [U * dvec(fw, fb, s) for s in range(n_long)], axis=0)
            return _ta(wl_ref[...], Gt.astype(bf)).astype(bf)

        Pt1 = spectral_pt(_ta(Xg, Vg), f1w, f1b, w1l_ref)
        for h in range(_NH):
            lo, hi = h * Hn, (h + 1) * Hn
            zab[F:2 * F, lo:hi] = _ta(Xg, Sg[:, lo:hi]).astype(bf)
        for h in range(_NH):
            lo, hi = h * Hn, (h + 1) * Hn
            zab[2 * F:3 * F, lo:hi] = _nn(zab[F:2 * F, :],
                                          Sg[:, lo:hi]).astype(bf)
        for h in range(_NH):
            lo, hi = h * Hn, (h + 1) * Hn
            acc = _tab(w1id_ref[...], Xg[lo:hi, :])
            acc = acc + _ta(w1sc_ref[...], zab[F:3 * F, lo:hi])
            acc = acc + _nn(Pt1, Vtg[:, lo:hi])
            zab[0:F, lo:hi] = jnp.maximum(acc + b1c, 0.0).astype(bf)

        Z1 = zab[0:F, :]
        Pt2 = spectral_pt(_nn(Z1, Vg), f2w, f2b, w2l_ref)
        for h in range(_NH):
            lo, hi = h * Hn, (h + 1) * Hn
            zab[F:2 * F, lo:hi] = _nn(Z1, Sg[:, lo:hi]).astype(bf)
        for h in range(_NH):
            lo, hi = h * Hn, (h + 1) * Hn
            zab[2 * F:3 * F, lo:hi] = _nn(zab[F:2 * F, :],
                                          Sg[:, lo:hi]).astype(bf)
        for h in range(_NH):
            lo, hi = h * Hn, (h + 1) * Hn
            acc = _ta(w2all_ref[...], zab[:, lo:hi])
            acc = acc + _nn(Pt2, Vtg[:, lo:hi])
            z2h = jnp.maximum(acc + b2c, 0.0).astype(bf)

            logits = _ta(z2h, wm_ref[...]) + bm_row
            logits = jnp.maximum(logits, 0.0)
            col = lax.broadcasted_iota(jnp.int32, logits.shape, 1)
            logits = jnp.where(col < nclass, logits, jnp.float32(-1e30))
            m = jnp.max(logits, axis=1, keepdims=True)
            sh = logits - m
            lsm = sh - jnp.log(jnp.sum(jnp.exp(sh), axis=1, keepdims=True))
            o_ref[g, lo:hi, :] = lsm[:, :nclass]


def kernel(X, S, V, Vt, R, W1s, W1l, b1, W2s, W2l, b2, Wm, bm,
           filt1_W, filt1_b, filt2_W, filt2_b):
    G, N, F0 = X.shape
    Kp = V.shape[-1]
    n_long = len(_LONG)
    nclass = _NCLASS
    F1 = b1.shape[1]
    F2 = b2.shape[1]
    gpb = _GPB if G % _GPB == 0 else 1

    R3 = R.reshape(G, 1, Kp).astype(jnp.float32)
    W1id = W1s[0]
    W1sc = W1s[1:].reshape(2 * F0, F1)
    W2all = W2s.reshape(3 * F1, F2)

    kern = functools.partial(_ln_kernel, n_long=n_long, nclass=nclass,
                             gpb=gpb)

    in_specs = [
        pl.BlockSpec((gpb, N, F0), lambda i: (i, 0, 0)),
        pl.BlockSpec((gpb, N, N), lambda i: (i, 0, 0)),
        pl.BlockSpec((gpb, N, Kp), lambda i: (i, 0, 0)),
        pl.BlockSpec((gpb, Kp, N), lambda i: (i, 0, 0)),
        pl.BlockSpec((gpb, 1, Kp), lambda i: (i, 0, 0)),
        pl.BlockSpec(W1id.shape, lambda i: (0, 0)),
        pl.BlockSpec(W1sc.shape, lambda i: (0, 0)),
        pl.BlockSpec(W1l.shape, lambda i: (0, 0)),
        pl.BlockSpec(b1.shape, lambda i: (0, 0)),
        pl.BlockSpec(W2all.shape, lambda i: (0, 0)),
        pl.BlockSpec(W2l.shape, lambda i: (0, 0)),
        pl.BlockSpec(b2.shape, lambda i: (0, 0)),
        pl.BlockSpec(Wm.shape, lambda i: (0, 0)),
        pl.BlockSpec(bm.shape, lambda i: (0, 0)),
        pl.BlockSpec(filt1_W.shape, lambda i: (0, 0)),
        pl.BlockSpec(filt1_b.shape, lambda i: (0, 0)),
        pl.BlockSpec(filt2_W.shape, lambda i: (0, 0)),
        pl.BlockSpec(filt2_b.shape, lambda i: (0, 0)),
    ]

    flops_layer = G * (2 * 2 * N * N * F1
                       + 2 * 3 * N * F1 * F1
                       + 2 * N * F1 * Kp * 2)
    cost = pl.CostEstimate(
        flops=int(2 * flops_layer + G * 2 * N * F2 * Wm.shape[1]),
        transcendentals=int(G * N * (Wm.shape[1] + 1)),
        bytes_accessed=int(X.size * 2 + S.size * 2 + 2 * V.size * 2
                           + G * N * nclass * 4))

    return pl.pallas_call(
        kern,
        out_shape=jax.ShapeDtypeStruct((G, N, nclass), jnp.float32),
        grid=(G // gpb,),
        in_specs=in_specs,
        out_specs=pl.BlockSpec((gpb, N, nclass), lambda i: (i, 0, 0)),
        scratch_shapes=[pltpu.VMEM((gpb, 3 * F1, N), jnp.bfloat16)],
        compiler_params=pltpu.CompilerParams(
            dimension_semantics=("parallel",)),
        cost_estimate=cost,
    )(X, S, V, Vt, R3, W1id, W1sc, W1l, b1, W2all, W2l, b2, Wm, bm,
      filt1_W, filt1_b, filt2_W, filt2_b)


# speedup vs baseline: 1.1754x; 1.1754x over previous
import functools

import jax
import jax.numpy as jnp
from jax import lax
from jax.experimental import pallas as pl
from jax.experimental.pallas import tpu as pltpu

_NCLASS = 64
_LONG = (2, 4)
_GPB = 4


def _ta(lhs, rhs):
    return lax.dot_general(lhs, rhs, (((0,), (0,)), ((), ())),
                           preferred_element_type=jnp.float32)


def _tab(lhs, rhs):
    return lax.dot_general(lhs, rhs, (((0,), (1,)), ((), ())),
                           preferred_element_type=jnp.float32)


def _nn(lhs, rhs):
    return jnp.dot(lhs, rhs, preferred_element_type=jnp.float32)


def _ln_kernel(x_ref, s_ref, v_ref, vt_ref, r_ref,
               w1id_ref, w1sc_ref, w1l_ref, b1_ref,
               w2all_ref, w2l_ref, b2_ref,
               wm_ref, bm_ref, f1w_ref, f1b_ref, f2w_ref, f2b_ref,
               o_ref, *, n_long, nclass, gpb):
    bf = jnp.bfloat16

    b1c = jnp.transpose(b1_ref[...], (1, 0))
    b2c = jnp.transpose(b2_ref[...], (1, 0))
    bm_row = bm_ref[...]
    f1w, f1b = f1w_ref[...], f1b_ref[...]
    f2w, f2b = f2w_ref[...], f2b_ref[...]

    for g in range(gpb):
        Xg = x_ref[g]
        Sg = s_ref[g]
        Vg = v_ref[g]
        Vtg = vt_ref[g]
        Rv = r_ref[g]

        Rp = {1: Rv}
        cur = Rv
        for t in range(2, max(_LONG) + 1):
            cur = cur * Rv
            if t in _LONG:
                Rp[t] = cur

        def dvec(fw, fb, s):
            d = fb[0:1, s:s + 1]
            for ti, t in enumerate(_LONG):
                d = d + fw[ti:ti + 1, s:s + 1] * Rp[t]
            return d

        def spectral_pt(U, fw, fb, wl_ref):
            Gt = jnp.concatenate(
                [U * dvec(fw, fb, s) for s in range(n_long)], axis=0)
            return _ta(wl_ref[...], Gt.astype(bf)).astype(bf)

        A1b = _ta(Xg, Sg).astype(bf)
        A2b = _nn(A1b, Sg).astype(bf)
        Pt1 = spectral_pt(_ta(Xg, Vg), f1w, f1b, w1l_ref)
        acc = _tab(w1id_ref[...], Xg)
        acc = acc + _ta(w1sc_ref[...],
                        jnp.concatenate([A1b, A2b], axis=0))
        acc = acc + _nn(Pt1, Vtg)
        Z1b = jnp.maximum(acc + b1c, 0.0).astype(bf)

        B1b = _nn(Z1b, Sg).astype(bf)
        B2b = _nn(B1b, Sg).astype(bf)
        Pt2 = spectral_pt(_nn(Z1b, Vg), f2w, f2b, w2l_ref)
        acc2 = _ta(w2all_ref[...],
                   jnp.concatenate([Z1b, B1b, B2b], axis=0))
        acc2 = acc2 + _nn(Pt2, Vtg)
        Z2b = jnp.maximum(acc2 + b2c, 0.0).astype(bf)

        logits = _ta(Z2b, wm_ref[...]) + bm_row
        logits = jnp.maximum(logits, 0.0)
        col = lax.broadcasted_iota(jnp.int32, logits.shape, 1)
        logits = jnp.where(col < nclass, logits, jnp.float32(-1e30))
        m = jnp.max(logits, axis=1, keepdims=True)
        sh = logits - m
        lsm = sh - jnp.log(jnp.sum(jnp.exp(sh), axis=1, keepdims=True))
        o_ref[g] = lsm[:, :nclass]


def kernel(X, S, V, Vt, R, W1s, W1l, b1, W2s, W2l, b2, Wm, bm,
           filt1_W, filt1_b, filt2_W, filt2_b):
    G, N, F0 = X.shape
    Kp = V.shape[-1]
    n_long = len(_LONG)
    nclass = _NCLASS
    F1 = b1.shape[1]
    F2 = b2.shape[1]
    gpb = _GPB if G % _GPB == 0 else 1

    R3 = R.reshape(G, 1, Kp).astype(jnp.float32)
    W1id = W1s[0]
    W1sc = W1s[1:].reshape(2 * F0, F1)
    W2all = W2s.reshape(3 * F1, F2)

    kern = functools.partial(_ln_kernel, n_long=n_long, nclass=nclass,
                             gpb=gpb)

    in_specs = [
        pl.BlockSpec((gpb, N, F0), lambda i: (i, 0, 0)),
        pl.BlockSpec((gpb, N, N), lambda i: (i, 0, 0)),
        pl.BlockSpec((gpb, N, Kp), lambda i: (i, 0, 0)),
        pl.BlockSpec((gpb, Kp, N), lambda i: (i, 0, 0)),
        pl.BlockSpec((gpb, 1, Kp), lambda i: (i, 0, 0)),
        pl.BlockSpec(W1id.shape, lambda i: (0, 0)),
        pl.BlockSpec(W1sc.shape, lambda i: (0, 0)),
        pl.BlockSpec(W1l.shape, lambda i: (0, 0)),
        pl.BlockSpec(b1.shape, lambda i: (0, 0)),
        pl.BlockSpec(W2all.shape, lambda i: (0, 0)),
        pl.BlockSpec(W2l.shape, lambda i: (0, 0)),
        pl.BlockSpec(b2.shape, lambda i: (0, 0)),
        pl.BlockSpec(Wm.shape, lambda i: (0, 0)),
        pl.BlockSpec(bm.shape, lambda i: (0, 0)),
        pl.BlockSpec(filt1_W.shape, lambda i: (0, 0)),
        pl.BlockSpec(filt1_b.shape, lambda i: (0, 0)),
        pl.BlockSpec(filt2_W.shape, lambda i: (0, 0)),
        pl.BlockSpec(filt2_b.shape, lambda i: (0, 0)),
    ]

    flops_layer = G * (2 * 2 * N * N * F1
                       + 2 * 3 * N * F1 * F1
                       + 2 * N * F1 * Kp * 2)
    cost = pl.CostEstimate(
        flops=int(2 * flops_layer + G * 2 * N * F2 * Wm.shape[1]),
        transcendentals=int(G * N * (Wm.shape[1] + 1)),
        bytes_accessed=int(X.size * 2 + S.size * 2 + 2 * V.size * 2
                           + G * N * nclass * 4))

    return pl.pallas_call(
        kern,
        out_shape=jax.ShapeDtypeStruct((G, N, nclass), jnp.float32),
        grid=(G // gpb,),
        in_specs=in_specs,
        out_specs=pl.BlockSpec((gpb, N, nclass), lambda i: (i, 0, 0)),
        compiler_params=pltpu.CompilerParams(
            dimension_semantics=("parallel",)),
        cost_estimate=cost,
    )(X, S, V, Vt, R3, W1id, W1sc, W1l, b1, W2all, W2l, b2, Wm, bm,
      filt1_W, filt1_b, filt2_W, filt2_b)


# speedup vs baseline: 1.2219x; 1.0395x over previous
import functools

import jax
import jax.numpy as jnp
from jax import lax
from jax.experimental import pallas as pl
from jax.experimental.pallas import tpu as pltpu

_NCLASS = 64
_LONG = (2, 4)
_GPB = 4


def _ta(lhs, rhs):
    return lax.dot_general(lhs, rhs, (((0,), (0,)), ((), ())),
                           preferred_element_type=jnp.float32)


def _tab(lhs, rhs):
    return lax.dot_general(lhs, rhs, (((0,), (1,)), ((), ())),
                           preferred_element_type=jnp.float32)


def _nn(lhs, rhs):
    return jnp.dot(lhs, rhs, preferred_element_type=jnp.float32)


def _ln_kernel(x_ref, s_ref, v_ref, vt_ref, r_ref,
               w1id_ref, w1sc_ref, w1l_ref, b1_ref,
               w2all_ref, w2l_ref, b2_ref,
               wm_ref, bm_ref, f1w_ref, f1b_ref, f2w_ref, f2b_ref,
               o_ref, *, n_long, nclass, gpb):
    bf = jnp.bfloat16

    b1c = jnp.transpose(b1_ref[...], (1, 0))
    b2c = jnp.transpose(b2_ref[...], (1, 0))
    bm_row = bm_ref[...]
    f1w, f1b = f1w_ref[...], f1b_ref[...]
    f2w, f2b = f2w_ref[...], f2b_ref[...]

    for g in range(gpb):
        Xg = x_ref[g]
        Sg = s_ref[g]
        Vg = v_ref[g]
        Vtg = vt_ref[g]
        Rv = r_ref[g]

        Rp = {1: Rv}
        cur = Rv
        for t in range(2, max(_LONG) + 1):
            cur = cur * Rv
            if t in _LONG:
                Rp[t] = cur

        def dvec(fw, fb, s):
            d = fb[0:1, s:s + 1]
            for ti, t in enumerate(_LONG):
                d = d + fw[ti:ti + 1, s:s + 1] * Rp[t]
            return d

        def spectral_pt(U, fw, fb, wl_ref):
            Gt = jnp.concatenate(
                [U * dvec(fw, fb, s) for s in range(n_long)], axis=0)
            return _ta(wl_ref[...], Gt.astype(bf)).astype(bf)

        A1b = _ta(Xg, Sg).astype(bf)
        A2b = _nn(A1b, Sg).astype(bf)
        Pt1 = spectral_pt(_ta(Xg, Vg), f1w, f1b, w1l_ref)
        acc = _tab(w1id_ref[...], Xg)
        acc = acc + _ta(w1sc_ref[0:128, :], A1b)
        acc = acc + _ta(w1sc_ref[128:256, :], A2b)
        acc = acc + _nn(Pt1, Vtg)
        Z1b = jnp.maximum(acc + b1c, 0.0).astype(bf)

        B1b = _nn(Z1b, Sg).astype(bf)
        B2b = _nn(B1b, Sg).astype(bf)
        Pt2 = spectral_pt(_nn(Z1b, Vg), f2w, f2b, w2l_ref)
        acc2 = _ta(w2all_ref[0:128, :], Z1b)
        acc2 = acc2 + _ta(w2all_ref[128:256, :], B1b)
        acc2 = acc2 + _ta(w2all_ref[256:384, :], B2b)
        acc2 = acc2 + _nn(Pt2, Vtg)
        Z2b = jnp.maximum(acc2 + b2c, 0.0).astype(bf)

        logits = _ta(Z2b, wm_ref[...]) + bm_row
        logits = jnp.maximum(logits, 0.0)
        col = lax.broadcasted_iota(jnp.int32, logits.shape, 1)
        logits = jnp.where(col < nclass, logits, jnp.float32(-1e30))
        m = jnp.max(logits, axis=1, keepdims=True)
        sh = logits - m
        lsm = sh - jnp.log(jnp.sum(jnp.exp(sh), axis=1, keepdims=True))
        o_ref[g] = lsm[:, :nclass]


def kernel(X, S, V, Vt, R, W1s, W1l, b1, W2s, W2l, b2, Wm, bm,
           filt1_W, filt1_b, filt2_W, filt2_b):
    G, N, F0 = X.shape
    Kp = V.shape[-1]
    n_long = len(_LONG)
    nclass = _NCLASS
    F1 = b1.shape[1]
    F2 = b2.shape[1]
    gpb = _GPB if G % _GPB == 0 else 1

    R3 = R.reshape(G, 1, Kp).astype(jnp.float32)
    W1id = W1s[0]
    W1sc = W1s[1:].reshape(2 * F0, F1)
    W2all = W2s.reshape(3 * F1, F2)

    kern = functools.partial(_ln_kernel, n_long=n_long, nclass=nclass,
                             gpb=gpb)

    in_specs = [
        pl.BlockSpec((gpb, N, F0), lambda i: (i, 0, 0)),
        pl.BlockSpec((gpb, N, N), lambda i: (i, 0, 0)),
        pl.BlockSpec((gpb, N, Kp), lambda i: (i, 0, 0)),
        pl.BlockSpec((gpb, Kp, N), lambda i: (i, 0, 0)),
        pl.BlockSpec((gpb, 1, Kp), lambda i: (i, 0, 0)),
        pl.BlockSpec(W1id.shape, lambda i: (0, 0)),
        pl.BlockSpec(W1sc.shape, lambda i: (0, 0)),
        pl.BlockSpec(W1l.shape, lambda i: (0, 0)),
        pl.BlockSpec(b1.shape, lambda i: (0, 0)),
        pl.BlockSpec(W2all.shape, lambda i: (0, 0)),
        pl.BlockSpec(W2l.shape, lambda i: (0, 0)),
        pl.BlockSpec(b2.shape, lambda i: (0, 0)),
        pl.BlockSpec(Wm.shape, lambda i: (0, 0)),
        pl.BlockSpec(bm.shape, lambda i: (0, 0)),
        pl.BlockSpec(filt1_W.shape, lambda i: (0, 0)),
        pl.BlockSpec(filt1_b.shape, lambda i: (0, 0)),
        pl.BlockSpec(filt2_W.shape, lambda i: (0, 0)),
        pl.BlockSpec(filt2_b.shape, lambda i: (0, 0)),
    ]

    flops_layer = G * (2 * 2 * N * N * F1
                       + 2 * 3 * N * F1 * F1
                       + 2 * N * F1 * Kp * 2)
    cost = pl.CostEstimate(
        flops=int(2 * flops_layer + G * 2 * N * F2 * Wm.shape[1]),
        transcendentals=int(G * N * (Wm.shape[1] + 1)),
        bytes_accessed=int(X.size * 2 + S.size * 2 + 2 * V.size * 2
                           + G * N * nclass * 4))

    return pl.pallas_call(
        kern,
        out_shape=jax.ShapeDtypeStruct((G, N, nclass), jnp.float32),
        grid=(G // gpb,),
        in_specs=in_specs,
        out_specs=pl.BlockSpec((gpb, N, nclass), lambda i: (i, 0, 0)),
        compiler_params=pltpu.CompilerParams(
            dimension_semantics=("arbitrary",)),
        cost_estimate=cost,
    )(X, S, V, Vt, R3, W1id, W1sc, W1l, b1, W2all, W2l, b2, Wm, bm,
      filt1_W, filt1_b, filt2_W, filt2_b)


# speedup vs baseline: 1.2253x; 1.0028x over previous
import functools

import jax
import jax.numpy as jnp
from jax import lax
from jax.experimental import pallas as pl
from jax.experimental.pallas import tpu as pltpu

_NCLASS = 64
_LONG = (2, 4)
_GPB = 4


def _ta(lhs, rhs):
    return lax.dot_general(lhs, rhs, (((0,), (0,)), ((), ())),
                           preferred_element_type=jnp.float32)


def _tab(lhs, rhs):
    return lax.dot_general(lhs, rhs, (((0,), (1,)), ((), ())),
                           preferred_element_type=jnp.float32)


def _nn(lhs, rhs):
    return jnp.dot(lhs, rhs, preferred_element_type=jnp.float32)


def _ln_kernel(x_ref, s_ref, v_ref, vt_ref, r_ref,
               w1id_ref, w1sc_ref, w1l_ref, b1_ref,
               w2all_ref, w2l_ref, b2_ref,
               wm_ref, bm_ref, f1w_ref, f1b_ref, f2w_ref, f2b_ref,
               o_ref, *, n_long, nclass, gpb):
    bf = jnp.bfloat16

    b1c = jnp.transpose(b1_ref[...], (1, 0))
    b2c = jnp.transpose(b2_ref[...], (1, 0))
    bm_row = bm_ref[...]
    f1w, f1b = f1w_ref[...], f1b_ref[...]
    f2w, f2b = f2w_ref[...], f2b_ref[...]

    for g in range(gpb):
        Xg = x_ref[g]
        Sg = s_ref[g]
        Vg = v_ref[g]
        Vtg = vt_ref[g]
        Rv = r_ref[g]

        Rp = {1: Rv}
        cur = Rv
        for t in range(2, max(_LONG) + 1):
            cur = cur * Rv
            if t in _LONG:
                Rp[t] = cur

        def dvec(fw, fb, s):
            d = fb[0:1, s:s + 1]
            for ti, t in enumerate(_LONG):
                d = d + fw[ti:ti + 1, s:s + 1] * Rp[t]
            return d

        def spectral_pt(U, fw, fb, wl_ref):
            Gt = jnp.concatenate(
                [U * dvec(fw, fb, s) for s in range(n_long)], axis=0)
            return _ta(wl_ref[...], Gt.astype(bf)).astype(bf)

        A1b = _ta(Xg, Sg).astype(bf)
        A2b = _nn(A1b, Sg).astype(bf)
        Pt1 = spectral_pt(_ta(Xg, Vg), f1w, f1b, w1l_ref)
        acc = _tab(w1id_ref[...], Xg)
        acc = acc + _ta(w1sc_ref[0:128, :], A1b)
        acc = acc + _ta(w1sc_ref[128:256, :], A2b)
        acc = acc + _nn(Pt1, Vtg)
        Z1b = jnp.maximum(acc + b1c, 0.0).astype(bf)

        B1b = _nn(Z1b, Sg).astype(bf)
        B2b = _nn(B1b, Sg).astype(bf)
        Pt2 = spectral_pt(_nn(Z1b, Vg), f2w, f2b, w2l_ref)
        acc2 = _ta(w2all_ref[0:128, :], Z1b)
        acc2 = acc2 + _ta(w2all_ref[128:256, :], B1b)
        acc2 = acc2 + _ta(w2all_ref[256:384, :], B2b)
        acc2 = acc2 + _nn(Pt2, Vtg)
        Z2b = jnp.maximum(acc2 + b2c, 0.0).astype(bf)

        logits = _ta(Z2b, wm_ref[...]) + bm_row
        logits = jnp.maximum(logits, 0.0)
        col = lax.broadcasted_iota(jnp.int32, logits.shape, 1)
        logits = jnp.where(col < nclass, logits, jnp.float32(-1e30))
        m = jnp.max(logits, axis=1, keepdims=True)
        sh = logits - m
        lsm = sh - jnp.log(jnp.sum(jnp.exp(sh), axis=1, keepdims=True))
        o_ref[g] = lsm[:, :nclass]


def kernel(X, S, V, Vt, R, W1s, W1l, b1, W2s, W2l, b2, Wm, bm,
           filt1_W, filt1_b, filt2_W, filt2_b):
    G, N, F0 = X.shape
    Kp = V.shape[-1]
    n_long = len(_LONG)
    nclass = _NCLASS
    F1 = b1.shape[1]
    F2 = b2.shape[1]
    gpb = _GPB if G % _GPB == 0 else 1

    R3 = R.reshape(G, 1, Kp).astype(jnp.float32)
    W1id = W1s[0]
    W1sc = W1s[1:].reshape(2 * F0, F1)
    W2all = W2s.reshape(3 * F1, F2)

    kern = functools.partial(_ln_kernel, n_long=n_long, nclass=nclass,
                             gpb=gpb)

    in_specs = [
        pl.BlockSpec((gpb, N, F0), lambda i: (i, 0, 0)),
        pl.BlockSpec((gpb, N, N), lambda i: (i, 0, 0)),
        pl.BlockSpec((gpb, N, Kp), lambda i: (i, 0, 0)),
        pl.BlockSpec((gpb, Kp, N), lambda i: (i, 0, 0)),
        pl.BlockSpec((gpb, 1, Kp), lambda i: (i, 0, 0)),
        pl.BlockSpec(W1id.shape, lambda i: (0, 0)),
        pl.BlockSpec(W1sc.shape, lambda i: (0, 0)),
        pl.BlockSpec(W1l.shape, lambda i: (0, 0)),
        pl.BlockSpec(b1.shape, lambda i: (0, 0)),
        pl.BlockSpec(W2all.shape, lambda i: (0, 0)),
        pl.BlockSpec(W2l.shape, lambda i: (0, 0)),
        pl.BlockSpec(b2.shape, lambda i: (0, 0)),
        pl.BlockSpec(Wm.shape, lambda i: (0, 0)),
        pl.BlockSpec(bm.shape, lambda i: (0, 0)),
        pl.BlockSpec(filt1_W.shape, lambda i: (0, 0)),
        pl.BlockSpec(filt1_b.shape, lambda i: (0, 0)),
        pl.BlockSpec(filt2_W.shape, lambda i: (0, 0)),
        pl.BlockSpec(filt2_b.shape, lambda i: (0, 0)),
    ]

    flops_layer = G * (2 * 2 * N * N * F1
                       + 2 * 3 * N * F1 * F1
                       + 2 * N * F1 * Kp * 2)
    cost = pl.CostEstimate(
        flops=int(2 * flops_layer + G * 2 * N * F2 * Wm.shape[1]),
        transcendentals=int(G * N * (Wm.shape[1] + 1)),
        bytes_accessed=int(X.size * 2 + S.size * 2 + 2 * V.size * 2
                           + G * N * nclass * 4))

    return pl.pallas_call(
        kern,
        out_shape=jax.ShapeDtypeStruct((G, N, nclass), jnp.float32),
        grid=(G // gpb,),
        in_specs=in_specs,
        out_specs=pl.BlockSpec((gpb, N, nclass), lambda i: (i, 0, 0)),
        compiler_params=pltpu.CompilerParams(
            dimension_semantics=("parallel",)),
        cost_estimate=cost,
    )(X, S, V, Vt, R3, W1id, W1sc, W1l, b1, W2all, W2l, b2, Wm, bm,
      filt1_W, filt1_b, filt2_W, filt2_b)


# speedup vs baseline: 1.2509x; 1.0209x over previous
import functools

import jax
import jax.numpy as jnp
from jax import lax
from jax.experimental import pallas as pl
from jax.experimental.pallas import tpu as pltpu

_NCLASS = 64
_LONG = (2, 4)
_GPB = 4


def _ta(lhs, rhs):
    return lax.dot_general(lhs, rhs, (((0,), (0,)), ((), ())),
                           preferred_element_type=jnp.float32)


def _tab(lhs, rhs):
    return lax.dot_general(lhs, rhs, (((0,), (1,)), ((), ())),
                           preferred_element_type=jnp.float32)


def _nn(lhs, rhs):
    return jnp.dot(lhs, rhs, preferred_element_type=jnp.float32)


def _ln_kernel(x_ref, s_ref, v_ref, vt_ref, r_ref,
               w1s_ref, w1l_ref, b1_ref,
               w2s_ref, w2l_ref, b2_ref,
               wm_ref, bm_ref, f1w_ref, f1b_ref, f2w_ref, f2b_ref,
               o_ref, *, n_long, nclass, gpb):
    bf = jnp.bfloat16

    b1c = jnp.transpose(b1_ref[...], (1, 0))
    b2c = jnp.transpose(b2_ref[...], (1, 0))
    bm_row = bm_ref[...]
    f1w, f1b = f1w_ref[...], f1b_ref[...]
    f2w, f2b = f2w_ref[...], f2b_ref[...]

    for g in range(gpb):
        Xg = x_ref[g]
        Sg = s_ref[g]
        Vg = v_ref[g]
        Vtg = vt_ref[g]
        Rv = r_ref[g]

        Rp = {1: Rv}
        cur = Rv
        for t in range(2, max(_LONG) + 1):
            cur = cur * Rv
            if t in _LONG:
                Rp[t] = cur

        def dvec(fw, fb, s):
            d = fb[0:1, s:s + 1]
            for ti, t in enumerate(_LONG):
                d = d + fw[ti:ti + 1, s:s + 1] * Rp[t]
            return d

        def spectral_pt(U, fw, fb, wl_ref):
            Gt = jnp.concatenate(
                [U * dvec(fw, fb, s) for s in range(n_long)], axis=0)
            return _ta(wl_ref[...], Gt.astype(bf)).astype(bf)

        A1b = _ta(Xg, Sg).astype(bf)
        A2b = _nn(A1b, Sg).astype(bf)
        Pt1 = spectral_pt(_ta(Xg, Vg), f1w, f1b, w1l_ref)
        acc = _tab(w1s_ref[0], Xg)
        acc = acc + _ta(w1s_ref[1], A1b)
        acc = acc + _ta(w1s_ref[2], A2b)
        acc = acc + _nn(Pt1, Vtg)
        Z1b = jnp.maximum(acc + b1c, 0.0).astype(bf)

        B1b = _nn(Z1b, Sg).astype(bf)
        B2b = _nn(B1b, Sg).astype(bf)
        Pt2 = spectral_pt(_nn(Z1b, Vg), f2w, f2b, w2l_ref)
        acc2 = _ta(w2s_ref[0], Z1b)
        acc2 = acc2 + _ta(w2s_ref[1], B1b)
        acc2 = acc2 + _ta(w2s_ref[2], B2b)
        acc2 = acc2 + _nn(Pt2, Vtg)
        Z2b = jnp.maximum(acc2 + b2c, 0.0).astype(bf)

        logits = _ta(Z2b, wm_ref[...]) + bm_row
        logits = jnp.maximum(logits, 0.0)
        col = lax.broadcasted_iota(jnp.int32, logits.shape, 1)
        logits = jnp.where(col < nclass, logits, jnp.float32(-1e30))
        m = jnp.max(logits, axis=1, keepdims=True)
        sh = logits - m
        lsm = sh - jnp.log(jnp.sum(jnp.exp(sh), axis=1, keepdims=True))
        o_ref[g] = lsm[:, :nclass]


def kernel(X, S, V, Vt, R, W1s, W1l, b1, W2s, W2l, b2, Wm, bm,
           filt1_W, filt1_b, filt2_W, filt2_b):
    G, N, F0 = X.shape
    Kp = V.shape[-1]
    n_long = len(_LONG)
    nclass = _NCLASS
    F1 = b1.shape[1]
    F2 = b2.shape[1]
    gpb = _GPB if G % _GPB == 0 else 1

    R3 = R.reshape(G, 1, Kp).astype(jnp.float32)

    kern = functools.partial(_ln_kernel, n_long=n_long, nclass=nclass,
                             gpb=gpb)

    in_specs = [
        pl.BlockSpec((gpb, N, F0), lambda i: (i, 0, 0)),
        pl.BlockSpec((gpb, N, N), lambda i: (i, 0, 0)),
        pl.BlockSpec((gpb, N, Kp), lambda i: (i, 0, 0)),
        pl.BlockSpec((gpb, Kp, N), lambda i: (i, 0, 0)),
        pl.BlockSpec((gpb, 1, Kp), lambda i: (i, 0, 0)),
        pl.BlockSpec(W1s.shape, lambda i: (0, 0, 0)),
        pl.BlockSpec(W1l.shape, lambda i: (0, 0)),
        pl.BlockSpec(b1.shape, lambda i: (0, 0)),
        pl.BlockSpec(W2s.shape, lambda i: (0, 0, 0)),
        pl.BlockSpec(W2l.shape, lambda i: (0, 0)),
        pl.BlockSpec(b2.shape, lambda i: (0, 0)),
        pl.BlockSpec(Wm.shape, lambda i: (0, 0)),
        pl.BlockSpec(bm.shape, lambda i: (0, 0)),
        pl.BlockSpec(filt1_W.shape, lambda i: (0, 0)),
        pl.BlockSpec(filt1_b.shape, lambda i: (0, 0)),
        pl.BlockSpec(filt2_W.shape, lambda i: (0, 0)),
        pl.BlockSpec(filt2_b.shape, lambda i: (0, 0)),
    ]

    flops_layer = G * (2 * 2 * N * N * F1
                       + 2 * 3 * N * F1 * F1
                       + 2 * N * F1 * Kp * 2)
    cost = pl.CostEstimate(
        flops=int(2 * flops_layer + G * 2 * N * F2 * Wm.shape[1]),
        transcendentals=int(G * N * (Wm.shape[1] + 1)),
        bytes_accessed=int(X.size * 2 + S.size * 2 + 2 * V.size * 2
                           + G * N * nclass * 4))

    return pl.pallas_call(
        kern,
        out_shape=jax.ShapeDtypeStruct((G, N, nclass), jnp.float32),
        grid=(G // gpb,),
        in_specs=in_specs,
        out_specs=pl.BlockSpec((gpb, N, nclass), lambda i: (i, 0, 0)),
        compiler_params=pltpu.CompilerParams(
            dimension_semantics=("parallel",)),
        cost_estimate=cost,
    )(X, S, V, Vt, R3, W1s, W1l, b1, W2s, W2l, b2, Wm, bm,
      filt1_W, filt1_b, filt2_W, filt2_b)


# speedup vs baseline: 1.2537x; 1.0023x over previous
import functools

import jax
import jax.numpy as jnp
from jax import lax
from jax.experimental import pallas as pl
from jax.experimental.pallas import tpu as pltpu

_NCLASS = 64
_LONG = (2, 4)
_GPB = 8


def _ta(lhs, rhs):
    return lax.dot_general(lhs, rhs, (((0,), (0,)), ((), ())),
                           preferred_element_type=jnp.float32)


def _tab(lhs, rhs):
    return lax.dot_general(lhs, rhs, (((0,), (1,)), ((), ())),
                           preferred_element_type=jnp.float32)


def _nn(lhs, rhs):
    return jnp.dot(lhs, rhs, preferred_element_type=jnp.float32)


def _ln_kernel(x_ref, s_ref, v_ref, vt_ref, r_ref,
               w1s_ref, w1l_ref, b1_ref,
               w2s_ref, w2l_ref, b2_ref,
               wm_ref, bm_ref, f1w_ref, f1b_ref, f2w_ref, f2b_ref,
               o_ref, *, n_long, nclass, gpb):
    bf = jnp.bfloat16

    b1c = jnp.transpose(b1_ref[...], (1, 0))
    b2c = jnp.transpose(b2_ref[...], (1, 0))
    bm_row = bm_ref[...]
    f1w, f1b = f1w_ref[...], f1b_ref[...]
    f2w, f2b = f2w_ref[...], f2b_ref[...]

    for g in range(gpb):
        Xg = x_ref[g]
        Sg = s_ref[g]
        Vg = v_ref[g]
        Vtg = vt_ref[g]
        Rv = r_ref[g]

        Rp = {1: Rv}
        cur = Rv
        for t in range(2, max(_LONG) + 1):
            cur = cur * Rv
            if t in _LONG:
                Rp[t] = cur

        def dvec(fw, fb, s):
            d = fb[0:1, s:s + 1]
            for ti, t in enumerate(_LONG):
                d = d + fw[ti:ti + 1, s:s + 1] * Rp[t]
            return d

        def spectral_pt(U, fw, fb, wl_ref):
            Gt = jnp.concatenate(
                [U * dvec(fw, fb, s) for s in range(n_long)], axis=0)
            return _ta(wl_ref[...], Gt.astype(bf)).astype(bf)

        A1b = _ta(Xg, Sg).astype(bf)
        A2b = _nn(A1b, Sg).astype(bf)
        Pt1 = spectral_pt(_ta(Xg, Vg), f1w, f1b, w1l_ref)
        acc = _tab(w1s_ref[0], Xg)
        acc = acc + _ta(w1s_ref[1], A1b)
        acc = acc + _ta(w1s_ref[2], A2b)
        acc = acc + _nn(Pt1, Vtg)
        Z1b = jnp.maximum(acc + b1c, 0.0).astype(bf)

        B1b = _nn(Z1b, Sg).astype(bf)
        B2b = _nn(B1b, Sg).astype(bf)
        Pt2 = spectral_pt(_nn(Z1b, Vg), f2w, f2b, w2l_ref)
        acc2 = _ta(w2s_ref[0], Z1b)
        acc2 = acc2 + _ta(w2s_ref[1], B1b)
        acc2 = acc2 + _ta(w2s_ref[2], B2b)
        acc2 = acc2 + _nn(Pt2, Vtg)
        Z2b = jnp.maximum(acc2 + b2c, 0.0).astype(bf)

        logits = _ta(Z2b, wm_ref[...]) + bm_row
        logits = jnp.maximum(logits, 0.0)
        col = lax.broadcasted_iota(jnp.int32, logits.shape, 1)
        logits = jnp.where(col < nclass, logits, jnp.float32(-1e30))
        m = jnp.max(logits, axis=1, keepdims=True)
        sh = logits - m
        lsm = sh - jnp.log(jnp.sum(jnp.exp(sh), axis=1, keepdims=True))
        o_ref[g] = lsm[:, :nclass]


def kernel(X, S, V, Vt, R, W1s, W1l, b1, W2s, W2l, b2, Wm, bm,
           filt1_W, filt1_b, filt2_W, filt2_b):
    G, N, F0 = X.shape
    Kp = V.shape[-1]
    n_long = len(_LONG)
    nclass = _NCLASS
    F1 = b1.shape[1]
    F2 = b2.shape[1]
    gpb = _GPB if G % _GPB == 0 else 1

    R3 = R.reshape(G, 1, Kp).astype(jnp.float32)

    kern = functools.partial(_ln_kernel, n_long=n_long, nclass=nclass,
                             gpb=gpb)

    in_specs = [
        pl.BlockSpec((gpb, N, F0), lambda i: (i, 0, 0)),
        pl.BlockSpec((gpb, N, N), lambda i: (i, 0, 0)),
        pl.BlockSpec((gpb, N, Kp), lambda i: (i, 0, 0)),
        pl.BlockSpec((gpb, Kp, N), lambda i: (i, 0, 0)),
        pl.BlockSpec((gpb, 1, Kp), lambda i: (i, 0, 0)),
        pl.BlockSpec(W1s.shape, lambda i: (0, 0, 0)),
        pl.BlockSpec(W1l.shape, lambda i: (0, 0)),
        pl.BlockSpec(b1.shape, lambda i: (0, 0)),
        pl.BlockSpec(W2s.shape, lambda i: (0, 0, 0)),
        pl.BlockSpec(W2l.shape, lambda i: (0, 0)),
        pl.BlockSpec(b2.shape, lambda i: (0, 0)),
        pl.BlockSpec(Wm.shape, lambda i: (0, 0)),
        pl.BlockSpec(bm.shape, lambda i: (0, 0)),
        pl.BlockSpec(filt1_W.shape, lambda i: (0, 0)),
        pl.BlockSpec(filt1_b.shape, lambda i: (0, 0)),
        pl.BlockSpec(filt2_W.shape, lambda i: (0, 0)),
        pl.BlockSpec(filt2_b.shape, lambda i: (0, 0)),
    ]

    flops_layer = G * (2 * 2 * N * N * F1
                       + 2 * 3 * N * F1 * F1
                       + 2 * N * F1 * Kp * 2)
    cost = pl.CostEstimate(
        flops=int(2 * flops_layer + G * 2 * N * F2 * Wm.shape[1]),
        transcendentals=int(G * N * (Wm.shape[1] + 1)),
        bytes_accessed=int(X.size * 2 + S.size * 2 + 2 * V.size * 2
                           + G * N * nclass * 4))

    return pl.pallas_call(
        kern,
        out_shape=jax.ShapeDtypeStruct((G, N, nclass), jnp.float32),
        grid=(G // gpb,),
        in_specs=in_specs,
        out_specs=pl.BlockSpec((gpb, N, nclass), lambda i: (i, 0, 0)),
        compiler_params=pltpu.CompilerParams(
            dimension_semantics=("parallel",)),
        cost_estimate=cost,
    )(X, S, V, Vt, R3, W1s, W1l, b1, W2s, W2l, b2, Wm, bm,
      filt1_W, filt1_b, filt2_W, filt2_b)


# speedup vs baseline: 1.2546x; 1.0007x over previous
import functools

import jax
import jax.numpy as jnp
from jax import lax
from jax.experimental import pallas as pl
from jax.experimental.pallas import tpu as pltpu

_NCLASS = 64
_LONG = (2, 4)
_GPB = 8


def _ta(lhs, rhs):
    return lax.dot_general(lhs, rhs, (((0,), (0,)), ((), ())),
                           preferred_element_type=jnp.float32)


def _tab(lhs, rhs):
    return lax.dot_general(lhs, rhs, (((0,), (1,)), ((), ())),
                           preferred_element_type=jnp.float32)


def _nn(lhs, rhs):
    return jnp.dot(lhs, rhs, preferred_element_type=jnp.float32)


def _ln_kernel(x_ref, s_ref, v_ref, vt_ref, r_ref,
               w1s_ref, w1l_ref, b1_ref,
               w2s_ref, w2l_ref, b2_ref,
               wm_ref, bm_ref, f1w_ref, f1b_ref, f2w_ref, f2b_ref,
               o_ref, *, n_long, nclass, gpb):
    bf = jnp.bfloat16

    b1c = jnp.transpose(b1_ref[...], (1, 0))
    b2c = jnp.transpose(b2_ref[...], (1, 0))
    bm_row = bm_ref[...]
    f1w, f1b = f1w_ref[...], f1b_ref[...]
    f2w, f2b = f2w_ref[...], f2b_ref[...]

    for g in range(gpb):
        Xg = x_ref[g]
        Sg = s_ref[g]
        Vg = v_ref[g]
        Vtg = vt_ref[g]
        Rv = r_ref[g]

        Rp = {1: Rv}
        cur = Rv
        for t in range(2, max(_LONG) + 1):
            cur = cur * Rv
            if t in _LONG:
                Rp[t] = cur

        def dvec(fw, fb, s):
            d = fb[0:1, s:s + 1]
            for ti, t in enumerate(_LONG):
                d = d + fw[ti:ti + 1, s:s + 1] * Rp[t]
            return d

        def spectral_pt(U, fw, fb, wl_ref):
            Gt = jnp.concatenate(
                [U * dvec(fw, fb, s) for s in range(n_long)], axis=0)
            return _ta(wl_ref[...], Gt.astype(bf)).astype(bf)

        A1b = _ta(Xg, Sg).astype(bf)
        A2b = _nn(A1b, Sg).astype(bf)
        Pt1 = spectral_pt(_ta(Xg, Vg), f1w, f1b, w1l_ref)
        acc = _tab(w1s_ref[0], Xg)
        acc = acc + _ta(w1s_ref[1], A1b)
        acc = acc + _ta(w1s_ref[2], A2b)
        acc = acc + _nn(Pt1, Vtg)
        Z1b = jnp.maximum(acc + b1c, 0.0).astype(bf)

        B1b = _nn(Z1b, Sg).astype(bf)
        B2b = _nn(B1b, Sg).astype(bf)
        Pt2 = spectral_pt(_nn(Z1b, Vg), f2w, f2b, w2l_ref)
        acc2 = _ta(w2s_ref[0], Z1b)
        acc2 = acc2 + _ta(w2s_ref[1], B1b)
        acc2 = acc2 + _ta(w2s_ref[2], B2b)
        acc2 = acc2 + _nn(Pt2, Vtg)
        Z2b = jnp.maximum(acc2 + b2c, 0.0).astype(bf)

        logits = _ta(Z2b, wm_ref[...]) + bm_row
        logits = jnp.maximum(logits[:, :nclass], 0.0)
        m = jnp.max(logits, axis=1, keepdims=True)
        sh = logits - m
        o_ref[g] = sh - jnp.log(jnp.sum(jnp.exp(sh), axis=1, keepdims=True))


def kernel(X, S, V, Vt, R, W1s, W1l, b1, W2s, W2l, b2, Wm, bm,
           filt1_W, filt1_b, filt2_W, filt2_b):
    G, N, F0 = X.shape
    Kp = V.shape[-1]
    n_long = len(_LONG)
    nclass = _NCLASS
    F1 = b1.shape[1]
    F2 = b2.shape[1]
    gpb = _GPB if G % _GPB == 0 else 1

    R3 = R.reshape(G, 1, Kp).astype(jnp.float32)

    kern = functools.partial(_ln_kernel, n_long=n_long, nclass=nclass,
                             gpb=gpb)

    in_specs = [
        pl.BlockSpec((gpb, N, F0), lambda i: (i, 0, 0)),
        pl.BlockSpec((gpb, N, N), lambda i: (i, 0, 0)),
        pl.BlockSpec((gpb, N, Kp), lambda i: (i, 0, 0)),
        pl.BlockSpec((gpb, Kp, N), lambda i: (i, 0, 0)),
        pl.BlockSpec((gpb, 1, Kp), lambda i: (i, 0, 0)),
        pl.BlockSpec(W1s.shape, lambda i: (0, 0, 0)),
        pl.BlockSpec(W1l.shape, lambda i: (0, 0)),
        pl.BlockSpec(b1.shape, lambda i: (0, 0)),
        pl.BlockSpec(W2s.shape, lambda i: (0, 0, 0)),
        pl.BlockSpec(W2l.shape, lambda i: (0, 0)),
        pl.BlockSpec(b2.shape, lambda i: (0, 0)),
        pl.BlockSpec(Wm.shape, lambda i: (0, 0)),
        pl.BlockSpec(bm.shape, lambda i: (0, 0)),
        pl.BlockSpec(filt1_W.shape, lambda i: (0, 0)),
        pl.BlockSpec(filt1_b.shape, lambda i: (0, 0)),
        pl.BlockSpec(filt2_W.shape, lambda i: (0, 0)),
        pl.BlockSpec(filt2_b.shape, lambda i: (0, 0)),
    ]

    flops_layer = G * (2 * 2 * N * N * F1
                       + 2 * 3 * N * F1 * F1
                       + 2 * N * F1 * Kp * 2)
    cost = pl.CostEstimate(
        flops=int(2 * flops_layer + G * 2 * N * F2 * Wm.shape[1]),
        transcendentals=int(G * N * (Wm.shape[1] + 1)),
        bytes_accessed=int(X.size * 2 + S.size * 2 + 2 * V.size * 2
                           + G * N * nclass * 4))

    return pl.pallas_call(
        kern,
        out_shape=jax.ShapeDtypeStruct((G, N, nclass), jnp.float32),
        grid=(G // gpb,),
        in_specs=in_specs,
        out_specs=pl.BlockSpec((gpb, N, nclass), lambda i: (i, 0, 0)),
        compiler_params=pltpu.CompilerParams(
            dimension_semantics=("parallel",)),
        cost_estimate=cost,
    )(X, S, V, Vt, R3, W1s, W1l, b1, W2s, W2l, b2, Wm, bm,
      filt1_W, filt1_b, filt2_W, filt2_b)
